# baseline (device time: 45665 ns/iter reference)
import jax
import jax.numpy as jnp
from jax import lax
from jax.experimental import pallas as pl
from jax.experimental.pallas import tpu as pltpu

N_DEV = 4
SQ = 1024
SKV = 1024
HQ_PER = 8
DH = 128
D_MODEL = 1024
SCALE = 0.08838834764831843


def kernel(x, Wq, K_ext, V_ext, Wo):
    my_pos = lax.axis_index("i")

    xb = x[0]
    wq_sl = lax.dynamic_slice(
        Wq.astype(jnp.bfloat16), (0, my_pos * HQ_PER * DH), (D_MODEL, HQ_PER * DH)
    )
    wo_sl = lax.dynamic_slice(
        Wo.astype(jnp.bfloat16), (my_pos * HQ_PER * DH, 0), (HQ_PER * DH, D_MODEL)
    )
    k = K_ext.reshape(SKV, HQ_PER * DH)
    v = V_ext.reshape(SKV, HQ_PER * DH)

    def body(x_ref, wq_ref, k_ref, v_ref, wo_ref, out_ref,
             rs_send_buf, rs_recv_buf, ag_send_buf, ag_recv_buf,
             rs_send_sems, rs_recv_sems, ag_send_sems, ag_recv_sems):
        my = lax.axis_index("i")

        barrier_sem = pltpu.get_barrier_semaphore()
        for d in range(1, N_DEV):
            pl.semaphore_signal(
                barrier_sem, inc=1,
                device_id=(lax.rem(my + d, N_DEV),),
                device_id_type=pl.DeviceIdType.MESH,
            )
        pl.semaphore_wait(barrier_sem, N_DEV - 1)

        CH = SQ // N_DEV

        def gather_group(ref, c):
            return jnp.concatenate(
                [ref[pl.ds((c + 4 * j) * 64, 64), :] for j in range(4)],
                axis=0).astype(jnp.bfloat16)

        def compute_chunk(c):
            xq = gather_group(x_ref, c)
            qc = jnp.dot(xq, wq_ref[...],
                         preferred_element_type=jnp.float32).astype(jnp.bfloat16)
            kc = gather_group(k_ref, c)
            vc = gather_group(v_ref, c)
            ctx_hs = []
            for h in range(HQ_PER):
                q_h = qc[:, h * DH:(h + 1) * DH]
                k_g = kc[:, h * DH:(h + 1) * DH]
                v_g = vc[:, h * DH:(h + 1) * DH]
                s = lax.dot_general(
                    q_h, k_g, (((1,), (1,)), ((), ())),
                    preferred_element_type=jnp.float32,
                ) * SCALE
                m = jnp.max(s, axis=-1, keepdims=True)
                w = jnp.exp(s - m)
                w = w / jnp.sum(w, axis=-1, keepdims=True)
                ctx_hs.append(
                    jnp.dot(w.astype(jnp.bfloat16), v_g,
                            preferred_element_type=jnp.float32)
                    .astype(jnp.bfloat16)
                )
            ctxc = jnp.concatenate(ctx_hs, axis=1)
            return jnp.dot(ctxc, wo_ref[...],
                           preferred_element_type=jnp.float32)

        rs_sends = []
        for d in range(1, N_DEV):
            peer = lax.rem(my + d, N_DEV)
            slot = d - 1
            dst_slot = (N_DEV - d) - 1
            rs_send_buf[slot, :, :] = compute_chunk(peer).astype(jnp.bfloat16)
            send = pltpu.make_async_remote_copy(
                src_ref=rs_send_buf.at[slot],
                dst_ref=rs_recv_buf.at[dst_slot],
                send_sem=rs_send_sems.at[slot],
                recv_sem=rs_recv_sems.at[dst_slot],
                device_id=(peer,),
                device_id_type=pl.DeviceIdType.MESH,
            )
            send.start()
            rs_sends.append(send)

        def store_chunk(c, val):
            for j in range(4):
                out_ref[0, pl.ds((c + 4 * j) * 64, 64), :] = val[
                    j * 64:(j + 1) * 64, :]

        acc = compute_chunk(my)
        for j in range(N_DEV - 1):
            recv = pltpu.make_async_remote_copy(
                src_ref=rs_send_buf.at[0],
                dst_ref=rs_recv_buf.at[j],
                send_sem=rs_send_sems.at[0],
                recv_sem=rs_recv_sems.at[j],
                device_id=(my,),
                device_id_type=pl.DeviceIdType.MESH,
            )
            recv.wait_recv()
            acc = acc + rs_recv_buf[j].astype(jnp.float32)

        acc = acc.astype(jnp.bfloat16)
        ag_send_buf[...] = acc
        ag_sends = []
        for d in range(1, N_DEV):
            peer = lax.rem(my + d, N_DEV)
            dst_slot = (N_DEV - d) - 1
            send = pltpu.make_async_remote_copy(
                src_ref=ag_send_buf,
                dst_ref=ag_recv_buf.at[dst_slot],
                send_sem=ag_send_sems.at[d - 1],
                recv_sem=ag_recv_sems.at[dst_slot],
                device_id=(peer,),
                device_id_type=pl.DeviceIdType.MESH,
            )
            send.start()
            ag_sends.append(send)

        store_chunk(my, acc)

        for j in range(N_DEV - 1):
            recv = pltpu.make_async_remote_copy(
                src_ref=ag_send_buf,
                dst_ref=ag_recv_buf.at[j],
                send_sem=ag_send_sems.at[0],
                recv_sem=ag_recv_sems.at[j],
                device_id=(my,),
                device_id_type=pl.DeviceIdType.MESH,
            )
            recv.wait_recv()
            c = lax.rem(my + j + 1, N_DEV)
            store_chunk(c, ag_recv_buf[j])

        for s in rs_sends + ag_sends:
            s.wait_send()

    out = pl.pallas_call(
        body,
        out_shape=jax.ShapeDtypeStruct((1, SQ, D_MODEL), jnp.bfloat16),
        in_specs=[pl.BlockSpec(memory_space=pltpu.VMEM)] * 5,
        out_specs=pl.BlockSpec(memory_space=pltpu.VMEM),
        scratch_shapes=[
            pltpu.VMEM((3, SQ // N_DEV, D_MODEL), jnp.bfloat16),
            pltpu.VMEM((3, SQ // N_DEV, D_MODEL), jnp.bfloat16),
            pltpu.VMEM((SQ // N_DEV, D_MODEL), jnp.bfloat16),
            pltpu.VMEM((3, SQ // N_DEV, D_MODEL), jnp.bfloat16),
            pltpu.SemaphoreType.DMA((3,)),
            pltpu.SemaphoreType.DMA((3,)),
            pltpu.SemaphoreType.DMA((3,)),
            pltpu.SemaphoreType.DMA((3,)),
        ],
        compiler_params=pltpu.CompilerParams(collective_id=0),
    )(xb, wq_sl, k, v, wo_sl)
    return out


# device time: 43422 ns/iter; 1.0517x vs baseline; 1.0517x over previous
import jax
import jax.numpy as jnp
from jax import lax
from jax.experimental import pallas as pl
from jax.experimental.pallas import tpu as pltpu

N_DEV = 4
SQ = 1024
SKV = 1024
HQ_PER = 8
DH = 128
D_MODEL = 1024
SCALE = 0.08838834764831843


def kernel(x, Wq, K_ext, V_ext, Wo):
    my_pos = lax.axis_index("i")

    xb = x[0]
    wq_sl = lax.dynamic_slice(
        Wq.astype(jnp.bfloat16), (0, my_pos * HQ_PER * DH), (D_MODEL, HQ_PER * DH)
    )
    wo_sl = lax.dynamic_slice(
        Wo.astype(jnp.bfloat16), (my_pos * HQ_PER * DH, 0), (HQ_PER * DH, D_MODEL)
    )
    k = K_ext.reshape(SKV, HQ_PER * DH)
    v = V_ext.reshape(SKV, HQ_PER * DH)

    def body(x_ref, wq_ref, k_ref, v_ref, wo_ref, out_ref,
             rs_send_buf, rs_recv_buf, ag_send_buf, ag_recv_buf,
             rs_send_sems, rs_recv_sems, ag_send_sems, ag_recv_sems):
        my = lax.axis_index("i")

        barrier_sem = pltpu.get_barrier_semaphore()
        for d in range(1, N_DEV):
            pl.semaphore_signal(
                barrier_sem, inc=1,
                device_id=(lax.rem(my + d, N_DEV),),
                device_id_type=pl.DeviceIdType.MESH,
            )
        pl.semaphore_wait(barrier_sem, N_DEV - 1)

        CH = SQ // N_DEV

        def gather_group(ref, c):
            return jnp.concatenate(
                [ref[pl.ds((c + 4 * j) * 64, 64), :] for j in range(4)],
                axis=0).astype(jnp.bfloat16)

        def compute_chunk(c):
            xq = gather_group(x_ref, c)
            qc = jnp.dot(xq, wq_ref[...],
                         preferred_element_type=jnp.float32).astype(jnp.bfloat16)
            kc = gather_group(k_ref, c)
            vc = gather_group(v_ref, c)
            ctx_hs = []
            for h in range(HQ_PER):
                q_h = qc[:, h * DH:(h + 1) * DH]
                k_g = kc[:, h * DH:(h + 1) * DH]
                v_g = vc[:, h * DH:(h + 1) * DH]
                s = lax.dot_general(
                    q_h, k_g, (((1,), (1,)), ((), ())),
                    preferred_element_type=jnp.float32,
                ) * SCALE
                m = jnp.max(s, axis=-1, keepdims=True)
                w = jnp.exp(s - m)
                w = w / jnp.sum(w, axis=-1, keepdims=True)
                ctx_hs.append(
                    jnp.dot(w.astype(jnp.bfloat16), v_g,
                            preferred_element_type=jnp.float32)
                    .astype(jnp.bfloat16)
                )
            ctxc = jnp.concatenate(ctx_hs, axis=1)
            return jnp.dot(ctxc, wo_ref[...],
                           preferred_element_type=jnp.float32)

        HC = D_MODEL // 2

        def rs_send(d, m):
            peer = lax.rem(my + d, N_DEV)
            send = pltpu.make_async_remote_copy(
                src_ref=rs_send_buf.at[d - 1, m],
                dst_ref=rs_recv_buf.at[(N_DEV - d) - 1, m],
                send_sem=rs_send_sems.at[d - 1, m],
                recv_sem=rs_recv_sems.at[(N_DEV - d) - 1, m],
                device_id=(peer,),
                device_id_type=pl.DeviceIdType.MESH,
            )
            send.start()
            return send

        sends = []
        for d in range(1, N_DEV):
            peer = lax.rem(my + d, N_DEV)
            chunk = compute_chunk(peer).astype(jnp.bfloat16)
            rs_send_buf[d - 1, 0, :, :] = chunk[:, :HC]
            rs_send_buf[d - 1, 1, :, :] = chunk[:, HC:]
            sends.append(rs_send(d, 0))

        own = compute_chunk(my)
        for d in range(1, N_DEV):
            sends.append(rs_send(d, 1))

        def store_chunk(c, val, m):
            for j in range(4):
                out_ref[0, pl.ds((c + 4 * j) * 64, 64),
                        m * HC:(m + 1) * HC] = val[j * 64:(j + 1) * 64, :]

        accs = [None, None]
        for m in (0, 1):
            acc = own[:, m * HC:(m + 1) * HC]
            for j in range(N_DEV - 1):
                recv = pltpu.make_async_remote_copy(
                    src_ref=rs_send_buf.at[0, m],
                    dst_ref=rs_recv_buf.at[j, m],
                    send_sem=rs_send_sems.at[0, m],
                    recv_sem=rs_recv_sems.at[j, m],
                    device_id=(my,),
                    device_id_type=pl.DeviceIdType.MESH,
                )
                recv.wait_recv()
                acc = acc + rs_recv_buf[j, m].astype(jnp.float32)
            acc = acc.astype(jnp.bfloat16)
            accs[m] = acc
            ag_send_buf[m, :, :] = acc
            for d in range(1, N_DEV):
                peer = lax.rem(my + d, N_DEV)
                send = pltpu.make_async_remote_copy(
                    src_ref=ag_send_buf.at[m],
                    dst_ref=ag_recv_buf.at[(N_DEV - d) - 1, m],
                    send_sem=ag_send_sems.at[d - 1, m],
                    recv_sem=ag_recv_sems.at[(N_DEV - d) - 1, m],
                    device_id=(peer,),
                    device_id_type=pl.DeviceIdType.MESH,
                )
                send.start()
                sends.append(send)

        for m in (0, 1):
            store_chunk(my, accs[m], m)

        for m in (0, 1):
            for j in range(N_DEV - 1):
                recv = pltpu.make_async_remote_copy(
                    src_ref=ag_send_buf.at[m],
                    dst_ref=ag_recv_buf.at[j, m],
                    send_sem=ag_send_sems.at[0, m],
                    recv_sem=ag_recv_sems.at[j, m],
                    device_id=(my,),
                    device_id_type=pl.DeviceIdType.MESH,
                )
                recv.wait_recv()
                c = lax.rem(my + j + 1, N_DEV)
                store_chunk(c, ag_recv_buf[j, m], m)

        for s in sends:
            s.wait_send()

    out = pl.pallas_call(
        body,
        out_shape=jax.ShapeDtypeStruct((1, SQ, D_MODEL), jnp.bfloat16),
        in_specs=[pl.BlockSpec(memory_space=pltpu.VMEM)] * 5,
        out_specs=pl.BlockSpec(memory_space=pltpu.VMEM),
        scratch_shapes=[
            pltpu.VMEM((3, 2, SQ // N_DEV, D_MODEL // 2), jnp.bfloat16),
            pltpu.VMEM((3, 2, SQ // N_DEV, D_MODEL // 2), jnp.bfloat16),
            pltpu.VMEM((2, SQ // N_DEV, D_MODEL // 2), jnp.bfloat16),
            pltpu.VMEM((3, 2, SQ // N_DEV, D_MODEL // 2), jnp.bfloat16),
            pltpu.SemaphoreType.DMA((3, 2)),
            pltpu.SemaphoreType.DMA((3, 2)),
            pltpu.SemaphoreType.DMA((3, 2)),
            pltpu.SemaphoreType.DMA((3, 2)),
        ],
        compiler_params=pltpu.CompilerParams(collective_id=0),
    )(xb, wq_sl, k, v, wo_sl)
    return out


# device time: 43254 ns/iter; 1.0557x vs baseline; 1.0039x over previous
import jax
import jax.numpy as jnp
from jax import lax
from jax.experimental import pallas as pl
from jax.experimental.pallas import tpu as pltpu

N_DEV = 4
SQ = 1024
SKV = 1024
HQ_PER = 8
DH = 128
D_MODEL = 1024
SCALE = 0.08838834764831843


def kernel(x, Wq, K_ext, V_ext, Wo):
    my_pos = lax.axis_index("i")

    xb = x[0]
    k = K_ext.reshape(SKV, HQ_PER * DH)
    v = V_ext.reshape(SKV, HQ_PER * DH)

    def body(x_ref, wq_ref, k_ref, v_ref, wo_ref, out_ref,
             wq_vmem, wo_vmem, cp_sems,
             rs_send_buf, rs_recv_buf, ag_send_buf, ag_recv_buf,
             rs_send_sems, rs_recv_sems, ag_send_sems, ag_recv_sems):
        my = lax.axis_index("i")

        wq_cp = pltpu.make_async_copy(
            wq_ref.at[:, pl.ds(my * HQ_PER * DH, HQ_PER * DH)],
            wq_vmem, cp_sems.at[0])
        wq_cp.start()
        wo_cp = pltpu.make_async_copy(
            wo_ref.at[pl.ds(my * HQ_PER * DH, HQ_PER * DH), :],
            wo_vmem, cp_sems.at[1])
        wo_cp.start()

        barrier_sem = pltpu.get_barrier_semaphore()
        for d in range(1, N_DEV):
            pl.semaphore_signal(
                barrier_sem, inc=1,
                device_id=(lax.rem(my + d, N_DEV),),
                device_id_type=pl.DeviceIdType.MESH,
            )
        pl.semaphore_wait(barrier_sem, N_DEV - 1)

        wq_cp.wait()
        wo_cp.wait()
        wq_b = wq_vmem[...].astype(jnp.bfloat16)
        wo_b = wo_vmem[...].astype(jnp.bfloat16)

        CH = SQ // N_DEV

        def gather_group(ref, c):
            return jnp.concatenate(
                [ref[pl.ds((c + 4 * j) * 64, 64), :] for j in range(4)],
                axis=0).astype(jnp.bfloat16)

        def compute_chunk(c):
            xq = gather_group(x_ref, c)
            qc = jnp.dot(xq, wq_b,
                         preferred_element_type=jnp.float32).astype(jnp.bfloat16)
            kc = gather_group(k_ref, c)
            vc = gather_group(v_ref, c)
            ctx_hs = []
            for h in range(HQ_PER):
                q_h = qc[:, h * DH:(h + 1) * DH]
                k_g = kc[:, h * DH:(h + 1) * DH]
                v_g = vc[:, h * DH:(h + 1) * DH]
                s = lax.dot_general(
                    q_h, k_g, (((1,), (1,)), ((), ())),
                    preferred_element_type=jnp.float32,
                ) * SCALE
                m = jnp.max(s, axis=-1, keepdims=True)
                w = jnp.exp(s - m)
                w = w / jnp.sum(w, axis=-1, keepdims=True)
                ctx_hs.append(
                    jnp.dot(w.astype(jnp.bfloat16), v_g,
                            preferred_element_type=jnp.float32)
                    .astype(jnp.bfloat16)
                )
            ctxc = jnp.concatenate(ctx_hs, axis=1)
            return jnp.dot(ctxc, wo_b,
                           preferred_element_type=jnp.float32)

        HC = D_MODEL // 2

        def rs_send(d, m):
            peer = lax.rem(my + d, N_DEV)
            send = pltpu.make_async_remote_copy(
                src_ref=rs_send_buf.at[d - 1, m],
                dst_ref=rs_recv_buf.at[(N_DEV - d) - 1, m],
                send_sem=rs_send_sems.at[d - 1, m],
                recv_sem=rs_recv_sems.at[(N_DEV - d) - 1, m],
                device_id=(peer,),
                device_id_type=pl.DeviceIdType.MESH,
            )
            send.start()
            return send

        sends = []
        for d in range(1, N_DEV):
            peer = lax.rem(my + d, N_DEV)
            chunk = compute_chunk(peer).astype(jnp.bfloat16)
            rs_send_buf[d - 1, 0, :, :] = chunk[:, :HC]
            rs_send_buf[d - 1, 1, :, :] = chunk[:, HC:]
            sends.append(rs_send(d, 0))

        own = compute_chunk(my)
        for d in range(1, N_DEV):
            sends.append(rs_send(d, 1))

        def store_chunk(c, val, m):
            for j in range(4):
                out_ref[0, pl.ds((c + 4 * j) * 64, 64),
                        m * HC:(m + 1) * HC] = val[j * 64:(j + 1) * 64, :]

        accs = [None, None]
        for m in (0, 1):
            acc = own[:, m * HC:(m + 1) * HC]
            for j in range(N_DEV - 1):
                recv = pltpu.make_async_remote_copy(
                    src_ref=rs_send_buf.at[0, m],
                    dst_ref=rs_recv_buf.at[j, m],
                    send_sem=rs_send_sems.at[0, m],
                    recv_sem=rs_recv_sems.at[j, m],
                    device_id=(my,),
                    device_id_type=pl.DeviceIdType.MESH,
                )
                recv.wait_recv()
                acc = acc + rs_recv_buf[j, m].astype(jnp.float32)
            acc = acc.astype(jnp.bfloat16)
            accs[m] = acc
            ag_send_buf[m, :, :] = acc
            for d in range(1, N_DEV):
                peer = lax.rem(my + d, N_DEV)
                send = pltpu.make_async_remote_copy(
                    src_ref=ag_send_buf.at[m],
                    dst_ref=ag_recv_buf.at[(N_DEV - d) - 1, m],
                    send_sem=ag_send_sems.at[d - 1, m],
                    recv_sem=ag_recv_sems.at[(N_DEV - d) - 1, m],
                    device_id=(peer,),
                    device_id_type=pl.DeviceIdType.MESH,
                )
                send.start()
                sends.append(send)

        for m in (0, 1):
            store_chunk(my, accs[m], m)

        for m in (0, 1):
            for j in range(N_DEV - 1):
                recv = pltpu.make_async_remote_copy(
                    src_ref=ag_send_buf.at[m],
                    dst_ref=ag_recv_buf.at[j, m],
                    send_sem=ag_send_sems.at[0, m],
                    recv_sem=ag_recv_sems.at[j, m],
                    device_id=(my,),
                    device_id_type=pl.DeviceIdType.MESH,
                )
                recv.wait_recv()
                c = lax.rem(my + j + 1, N_DEV)
                store_chunk(c, ag_recv_buf[j, m], m)

        for s in sends:
            s.wait_send()

    out = pl.pallas_call(
        body,
        out_shape=jax.ShapeDtypeStruct((1, SQ, D_MODEL), jnp.bfloat16),
        in_specs=[
            pl.BlockSpec(memory_space=pltpu.VMEM),
            pl.BlockSpec(memory_space=pl.ANY),
            pl.BlockSpec(memory_space=pltpu.VMEM),
            pl.BlockSpec(memory_space=pltpu.VMEM),
            pl.BlockSpec(memory_space=pl.ANY),
        ],
        out_specs=pl.BlockSpec(memory_space=pltpu.VMEM),
        scratch_shapes=[
            pltpu.VMEM((D_MODEL, HQ_PER * DH), jnp.float32),
            pltpu.VMEM((HQ_PER * DH, D_MODEL), jnp.float32),
            pltpu.SemaphoreType.DMA((2,)),
            pltpu.VMEM((3, 2, SQ // N_DEV, D_MODEL // 2), jnp.bfloat16),
            pltpu.VMEM((3, 2, SQ // N_DEV, D_MODEL // 2), jnp.bfloat16),
            pltpu.VMEM((2, SQ // N_DEV, D_MODEL // 2), jnp.bfloat16),
            pltpu.VMEM((3, 2, SQ // N_DEV, D_MODEL // 2), jnp.bfloat16),
            pltpu.SemaphoreType.DMA((3, 2)),
            pltpu.SemaphoreType.DMA((3, 2)),
            pltpu.SemaphoreType.DMA((3, 2)),
            pltpu.SemaphoreType.DMA((3, 2)),
        ],
        compiler_params=pltpu.CompilerParams(collective_id=0),
    )(xb, Wq, k, v, Wo)
    return out
